# Initial kernel scaffold; baseline (speedup 1.0000x reference)
import jax
import jax.numpy as jnp
from jax.experimental import pallas as pl


def kernel(y_s, y_t):
    raise NotImplementedError("write your pallas kernel here")


# speedup vs baseline: 3.8260x; 3.8260x over previous
import functools

import jax
import jax.numpy as jnp
from jax import lax
from jax.experimental import pallas as pl
from jax.experimental.pallas import tpu as pltpu


def _cmpx_chunk(c, c0, size, stride, rc, lanes):
  if stride >= 8:
    g = rc // (2 * stride)
    r = c.reshape(g, 2, stride, lanes)
    lo = r[:, 0]
    hi = r[:, 1]
    mn = jnp.minimum(lo, hi)
    mx = jnp.maximum(lo, hi)
    shift = size.bit_length() - stride.bit_length() - 1
    gi = lax.broadcasted_iota(jnp.int32, (g, 1, 1), 0) + c0 // (2 * stride)
    asc = ((gi >> shift) & 1) == 0
    new_lo = jnp.where(asc, mn, mx)
    new_hi = jnp.where(asc, mx, mn)
    return jnp.concatenate([new_lo[:, None], new_hi[:, None]], axis=1).reshape(
        rc, lanes
    )
  else:
    gi = lax.broadcasted_iota(jnp.int32, (rc, 1), 0) + c0
    y = jnp.concatenate([c[stride:], c[:stride]], axis=0)
    z = jnp.concatenate([c[-stride:], c[:-stride]], axis=0)
    low = (gi & stride) == 0
    asc = (gi & size) == 0
    lo_res = jnp.where(asc, jnp.minimum(c, y), jnp.maximum(c, y))
    hi_res = jnp.where(asc, jnp.maximum(c, z), jnp.minimum(c, z))
    return jnp.where(low, lo_res, hi_res)


def _sort_mse_body(x_hbm, o_ref, x_vmem, sem, *, n_rows, lanes, cb, rc):
  b = pl.program_id(0)
  copy = pltpu.make_async_copy(
      x_hbm.at[:, pl.ds(b * lanes, lanes)], x_vmem, sem
  )
  copy.start()
  copy.wait()

  nbits = n_rows.bit_length() - 1
  for ks in range(1, nbits + 1):
    size = 1 << ks
    for js in range(ks - 1, -1, -1):
      stride = 1 << js
      if 2 * stride <= rc:
        def f_small(t, _, size=size, stride=stride):
          c0 = t * rc
          c = x_vmem[pl.ds(c0, rc), :]
          x_vmem[pl.ds(c0, rc), :] = _cmpx_chunk(c, c0, size, stride, rc, lanes)
          return 0

        lax.fori_loop(0, n_rows // rc, f_small, 0)
      else:
        per_group = stride // rc

        def f_big(t, _, size=size, stride=stride, per_group=per_group):
          g = t // per_group
          off = (t % per_group) * rc
          base = g * (2 * stride)
          a0 = base + off
          a = x_vmem[pl.ds(a0, rc), :]
          b_ = x_vmem[pl.ds(a0 + stride, rc), :]
          mn = jnp.minimum(a, b_)
          mx = jnp.maximum(a, b_)
          asc = (base & size) == 0
          x_vmem[pl.ds(a0, rc), :] = jnp.where(asc, mn, mx)
          x_vmem[pl.ds(a0 + stride, rc), :] = jnp.where(asc, mx, mn)
          return 0

        lax.fori_loop(0, (n_rows // 2) // rc, f_big, 0)

  def f_mse(t, s):
    c = x_vmem[pl.ds(t * rc, rc), :]
    d = c[:, :cb] - c[:, cb:]
    return s + jnp.sum(d * d)

  total = lax.fori_loop(0, n_rows // rc, f_mse, jnp.float32(0.0))
  o_ref[...] = jnp.full((1, 8, 128), total, jnp.float32)


@jax.jit
def kernel(y_s, y_t):
  n_rows, c = y_s.shape
  assert n_rows & (n_rows - 1) == 0, "rows must be a power of two"
  cb = 64 if c % 64 == 0 else c
  grid = c // cb
  lanes = 2 * cb
  rc = min(2048, n_rows)
  x_all = jnp.concatenate(
      [y_s.reshape(n_rows, grid, cb), y_t.reshape(n_rows, grid, cb)], axis=2
  ).reshape(n_rows, 2 * c)

  body = functools.partial(
      _sort_mse_body, n_rows=n_rows, lanes=lanes, cb=cb, rc=rc
  )
  partials = pl.pallas_call(
      body,
      grid=(grid,),
      in_specs=[pl.BlockSpec(memory_space=pl.ANY)],
      out_specs=pl.BlockSpec((1, 8, 128), lambda b: (b, 0, 0)),
      out_shape=jax.ShapeDtypeStruct((grid, 8, 128), jnp.float32),
      scratch_shapes=[
          pltpu.VMEM((n_rows, lanes), jnp.float32),
          pltpu.SemaphoreType.DMA,
      ],
      compiler_params=pltpu.CompilerParams(
          dimension_semantics=("arbitrary",),
      ),
  )(x_all)
  return jnp.sum(partials[:, 0, 0]) / (n_rows * c)


# speedup vs baseline: 5.5569x; 1.4524x over previous
import functools

import jax
import jax.numpy as jnp
from jax import lax
from jax.experimental import pallas as pl
from jax.experimental.pallas import tpu as pltpu


def _cmpx_chunk(c, c0, size, stride, rc, lanes):
  if stride >= 8:
    g = rc // (2 * stride)
    r = c.reshape(g, 2, stride, lanes)
    lo = r[:, 0]
    hi = r[:, 1]
    mn = jnp.minimum(lo, hi)
    mx = jnp.maximum(lo, hi)
    shift = size.bit_length() - stride.bit_length() - 1
    gi = lax.broadcasted_iota(jnp.int32, (g, 1, 1), 0) + c0 // (2 * stride)
    asc = ((gi >> shift) & 1) == 0
    new_lo = jnp.where(asc, mn, mx)
    new_hi = jnp.where(asc, mx, mn)
    return jnp.concatenate([new_lo[:, None], new_hi[:, None]], axis=1).reshape(
        rc, lanes
    )
  else:
    gi = lax.broadcasted_iota(jnp.int32, (rc, 1), 0) + c0
    y = jnp.concatenate([c[stride:], c[:stride]], axis=0)
    z = jnp.concatenate([c[-stride:], c[:-stride]], axis=0)
    low = (gi & stride) == 0
    asc = (gi & size) == 0
    lo_res = jnp.where(asc, jnp.minimum(c, y), jnp.maximum(c, y))
    hi_res = jnp.where(asc, jnp.maximum(c, z), jnp.minimum(c, z))
    return jnp.where(low, lo_res, hi_res)


def _sort_mse_body(x_hbm, o_ref, x_vmem, sem, *, n_rows, lanes, cb, rc):
  b = pl.program_id(0)
  copy = pltpu.make_async_copy(
      x_hbm.at[:, pl.ds(b * lanes, lanes)], x_vmem, sem
  )
  copy.start()
  copy.wait()

  fc = min(64, n_rows)

  def fused_small_stage(ks):
    size = 1 << ks
    strides = [1 << j for j in range(min(ks - 1, fc.bit_length() - 2), -1, -1)]
    li = lax.broadcasted_iota(jnp.int32, (fc, 1), 0)

    def f(t, _):
      c0 = t * fc
      c = x_vmem[pl.ds(c0, fc), :]
      if size >= fc:
        flip = jnp.where((c0 & size) == 0, jnp.float32(1.0), jnp.float32(-1.0))
      else:
        flip = jnp.where((li & size) == 0, jnp.float32(1.0), jnp.float32(-1.0))
      c = c * flip
      for s in strides:
        if s >= 8:
          g = fc // (2 * s)
          r = c.reshape(g, 2, s, lanes)
          mn = jnp.minimum(r[:, 0], r[:, 1])
          mx = jnp.maximum(r[:, 0], r[:, 1])
          c = jnp.concatenate([mn[:, None], mx[:, None]], axis=1).reshape(
              fc, lanes
          )
        else:
          y = jnp.concatenate([c[s:], c[:s]], axis=0)
          z = jnp.concatenate([c[-s:], c[:-s]], axis=0)
          c = jnp.where(
              (li & s) == 0, jnp.minimum(c, y), jnp.maximum(c, z)
          )
      c = c * flip
      x_vmem[pl.ds(c0, fc), :] = c
      return 0

    lax.fori_loop(0, n_rows // fc, f, 0)

  nbits = n_rows.bit_length() - 1
  for ks in range(1, nbits + 1):
    size = 1 << ks
    for js in range(ks - 1, -1, -1):
      stride = 1 << js
      if 2 * stride <= fc:
        fused_small_stage(ks)
        break
      elif 2 * stride <= rc:
        def f_small(t, _, size=size, stride=stride):
          c0 = t * rc
          c = x_vmem[pl.ds(c0, rc), :]
          x_vmem[pl.ds(c0, rc), :] = _cmpx_chunk(c, c0, size, stride, rc, lanes)
          return 0

        lax.fori_loop(0, n_rows // rc, f_small, 0)
      else:
        per_group = stride // rc

        def f_big(t, _, size=size, stride=stride, per_group=per_group):
          g = t // per_group
          off = (t % per_group) * rc
          base = g * (2 * stride)
          a0 = base + off
          a = x_vmem[pl.ds(a0, rc), :]
          b_ = x_vmem[pl.ds(a0 + stride, rc), :]
          mn = jnp.minimum(a, b_)
          mx = jnp.maximum(a, b_)
          asc = (base & size) == 0
          x_vmem[pl.ds(a0, rc), :] = jnp.where(asc, mn, mx)
          x_vmem[pl.ds(a0 + stride, rc), :] = jnp.where(asc, mx, mn)
          return 0

        lax.fori_loop(0, (n_rows // 2) // rc, f_big, 0)

  def f_mse(t, s):
    c = x_vmem[pl.ds(t * rc, rc), :]
    d = c[:, :cb] - c[:, cb:]
    return s + jnp.sum(d * d)

  total = lax.fori_loop(0, n_rows // rc, f_mse, jnp.float32(0.0))
  o_ref[...] = jnp.full((1, 8, 128), total, jnp.float32)


@jax.jit
def kernel(y_s, y_t):
  n_rows, c = y_s.shape
  assert n_rows & (n_rows - 1) == 0, "rows must be a power of two"
  cb = 64 if c % 64 == 0 else c
  grid = c // cb
  lanes = 2 * cb
  rc = min(2048, n_rows)
  x_all = jnp.concatenate(
      [y_s.reshape(n_rows, grid, cb), y_t.reshape(n_rows, grid, cb)], axis=2
  ).reshape(n_rows, 2 * c)

  body = functools.partial(
      _sort_mse_body, n_rows=n_rows, lanes=lanes, cb=cb, rc=rc
  )
  partials = pl.pallas_call(
      body,
      grid=(grid,),
      in_specs=[pl.BlockSpec(memory_space=pl.ANY)],
      out_specs=pl.BlockSpec((1, 8, 128), lambda b: (b, 0, 0)),
      out_shape=jax.ShapeDtypeStruct((grid, 8, 128), jnp.float32),
      scratch_shapes=[
          pltpu.VMEM((n_rows, lanes), jnp.float32),
          pltpu.SemaphoreType.DMA,
      ],
      compiler_params=pltpu.CompilerParams(
          dimension_semantics=("arbitrary",),
      ),
  )(x_all)
  return jnp.sum(partials[:, 0, 0]) / (n_rows * c)


# speedup vs baseline: 5.9725x; 1.0748x over previous
import functools

import jax
import jax.numpy as jnp
from jax import lax
from jax.experimental import pallas as pl
from jax.experimental.pallas import tpu as pltpu


def _sort_mse_body(x_hbm, o_ref, x_vmem, sem, *, n_rows, lanes, cb, fs, bc):
  b = pl.program_id(0)
  copy = pltpu.make_async_copy(
      x_hbm.at[:, pl.ds(b * lanes, lanes)], x_vmem, sem
  )
  copy.start()
  copy.wait()

  def fused_small_stage(ks):
    size = 1 << ks
    strides = [1 << j for j in range(min(ks - 1, fs.bit_length() - 2), -1, -1)]
    li = lax.broadcasted_iota(jnp.int32, (fs, 1), 0)

    def f(t, _):
      c0 = t * fs
      c = x_vmem[pl.ds(c0, fs), :]
      if size >= fs:
        flip = jnp.where((c0 & size) == 0, jnp.float32(1.0), jnp.float32(-1.0))
      else:
        flip = jnp.where((li & size) == 0, jnp.float32(1.0), jnp.float32(-1.0))
      c = c * flip
      for s in strides:
        if s >= 8:
          g = fs // (2 * s)
          r = c.reshape(g, 2, s, lanes)
          mn = jnp.minimum(r[:, 0], r[:, 1])
          mx = jnp.maximum(r[:, 0], r[:, 1])
          c = jnp.concatenate([mn[:, None], mx[:, None]], axis=1).reshape(
              fs, lanes
          )
        else:
          y = jnp.concatenate([c[s:], c[:s]], axis=0)
          z = jnp.concatenate([c[-s:], c[:-s]], axis=0)
          c = jnp.where((li & s) == 0, jnp.minimum(c, y), jnp.maximum(c, z))
      c = c * flip
      x_vmem[pl.ds(c0, fs), :] = c
      return 0

    lax.fori_loop(0, n_rows // fs, f, 0)

  def pair_pass(size, s):
    p = (s // bc).bit_length() - 1

    def f(t, _):
      q = ((t >> p) << (p + 1)) | (t & ((1 << p) - 1))
      b0 = q * bc
      asc = (b0 & size) == 0
      a = x_vmem[pl.ds(b0, bc), :]
      d = x_vmem[pl.ds(b0 + s, bc), :]
      mn = jnp.minimum(a, d)
      mx = jnp.maximum(a, d)
      x_vmem[pl.ds(b0, bc), :] = jnp.where(asc, mn, mx)
      x_vmem[pl.ds(b0 + s, bc), :] = jnp.where(asc, mx, mn)
      return 0

    lax.fori_loop(0, n_rows // (2 * bc), f, 0)

  def quad_pass(size, s1, s2):
    p2 = (s2 // bc).bit_length() - 1

    def f(t, _):
      q = ((t >> p2) << (p2 + 2)) | (t & ((1 << p2) - 1))
      b0 = q * bc
      asc = (b0 & size) == 0
      a = x_vmem[pl.ds(b0, bc), :]
      b_ = x_vmem[pl.ds(b0 + s2, bc), :]
      c = x_vmem[pl.ds(b0 + s1, bc), :]
      d = x_vmem[pl.ds(b0 + s1 + s2, bc), :]
      mn, mx = jnp.minimum(a, c), jnp.maximum(a, c)
      a1, c1 = jnp.where(asc, mn, mx), jnp.where(asc, mx, mn)
      mn, mx = jnp.minimum(b_, d), jnp.maximum(b_, d)
      b1, d1 = jnp.where(asc, mn, mx), jnp.where(asc, mx, mn)
      mn, mx = jnp.minimum(a1, b1), jnp.maximum(a1, b1)
      x_vmem[pl.ds(b0, bc), :] = jnp.where(asc, mn, mx)
      x_vmem[pl.ds(b0 + s2, bc), :] = jnp.where(asc, mx, mn)
      mn, mx = jnp.minimum(c1, d1), jnp.maximum(c1, d1)
      x_vmem[pl.ds(b0 + s1, bc), :] = jnp.where(asc, mn, mx)
      x_vmem[pl.ds(b0 + s1 + s2, bc), :] = jnp.where(asc, mx, mn)
      return 0

    lax.fori_loop(0, n_rows // (4 * bc), f, 0)

  nbits = n_rows.bit_length() - 1
  for ks in range(1, nbits + 1):
    size = 1 << ks
    bigs = [1 << j for j in range(ks - 1, fs.bit_length() - 2, -1)]
    i = 0
    while i < len(bigs):
      if i + 1 < len(bigs):
        quad_pass(size, bigs[i], bigs[i + 1])
        i += 2
      else:
        pair_pass(size, bigs[i])
        i += 1
    fused_small_stage(ks)

  rm = min(2048, n_rows)

  def f_mse(t, s):
    c = x_vmem[pl.ds(t * rm, rm), :]
    d = c[:, :cb] - c[:, cb:]
    return s + jnp.sum(d * d)

  total = lax.fori_loop(0, n_rows // rm, f_mse, jnp.float32(0.0))
  o_ref[...] = jnp.full((1, 8, 128), total, jnp.float32)


@jax.jit
def kernel(y_s, y_t):
  n_rows, c = y_s.shape
  assert n_rows & (n_rows - 1) == 0, "rows must be a power of two"
  cb = 64 if c % 64 == 0 else c
  grid = c // cb
  lanes = 2 * cb
  fs = min(128, n_rows)
  bc = min(64, n_rows)
  x_all = jnp.concatenate(
      [y_s.reshape(n_rows, grid, cb), y_t.reshape(n_rows, grid, cb)], axis=2
  ).reshape(n_rows, 2 * c)

  body = functools.partial(
      _sort_mse_body, n_rows=n_rows, lanes=lanes, cb=cb, fs=fs, bc=bc
  )
  partials = pl.pallas_call(
      body,
      grid=(grid,),
      in_specs=[pl.BlockSpec(memory_space=pl.ANY)],
      out_specs=pl.BlockSpec((1, 8, 128), lambda b: (b, 0, 0)),
      out_shape=jax.ShapeDtypeStruct((grid, 8, 128), jnp.float32),
      scratch_shapes=[
          pltpu.VMEM((n_rows, lanes), jnp.float32),
          pltpu.SemaphoreType.DMA,
      ],
      compiler_params=pltpu.CompilerParams(
          dimension_semantics=("arbitrary",),
      ),
  )(x_all)
  return jnp.sum(partials[:, 0, 0]) / (n_rows * c)


# speedup vs baseline: 6.5562x; 1.0977x over previous
import functools

import jax
import jax.numpy as jnp
from jax import lax
from jax.experimental import pallas as pl
from jax.experimental.pallas import tpu as pltpu


def _insert_bit(t, pos, val):
  out = ((t >> pos) << (pos + 1)) | (t & ((1 << pos) - 1))
  if val:
    out = out | (1 << pos)
  return out


def _log2(v):
  return v.bit_length() - 1


def _sort_mse_body(x_hbm, o_ref, x_vmem, sem, *, n_rows, lanes, cb, fs, bc):
  b = pl.program_id(0)
  copy = pltpu.make_async_copy(
      x_hbm.at[:, pl.ds(b * lanes, lanes)], x_vmem, sem
  )
  copy.start()
  copy.wait()

  nbits = _log2(n_rows)
  li = lax.broadcasted_iota(jnp.int32, (fs, 1), 0)
  small_masks = {s: (li & s) == 0 for s in (1, 2, 4) if s < fs}
  flip_vecs = {
      ks: jnp.where(
          (li & (1 << ks)) == 0, jnp.float32(1.0), jnp.float32(-1.0)
      )
      for ks in range(1, min(nbits, _log2(fs) - 1) + 1)
  }

  def network(c, strides, desc):
    for s in strides:
      if s >= 8:
        g = fs // (2 * s)
        r = c.reshape(g, 2, s, lanes)
        mn = jnp.minimum(r[:, 0], r[:, 1])
        mx = jnp.maximum(r[:, 0], r[:, 1])
        pair = [mx[:, None], mn[:, None]] if desc else [mn[:, None], mx[:, None]]
        c = jnp.concatenate(pair, axis=1).reshape(fs, lanes)
      else:
        y = jnp.concatenate([c[s:], c[:s]], axis=0)
        z = jnp.concatenate([c[-s:], c[:-s]], axis=0)
        if desc:
          c = jnp.where(small_masks[s], jnp.maximum(c, y), jnp.minimum(c, z))
        else:
          c = jnp.where(small_masks[s], jnp.minimum(c, y), jnp.maximum(c, z))
    return c

  def fused_small_stage(ks):
    size = 1 << ks
    strides = [1 << j for j in range(min(ks - 1, _log2(fs) - 1), -1, -1)]
    nchunks = n_rows // fs

    if size >= fs:
      ps = _log2(size // fs)
      nqb = _log2(nchunks)

      def make_f(desc):
        def f(t, _):
          q = t if ps >= nqb else _insert_bit(t, ps, 1 if desc else 0)
          c0 = q * fs
          c = x_vmem[pl.ds(c0, fs), :]
          x_vmem[pl.ds(c0, fs), :] = network(c, strides, desc)
          return 0

        return f

      if ps >= nqb:
        lax.fori_loop(0, nchunks, make_f(False), 0)
      else:
        lax.fori_loop(0, nchunks // 2, make_f(False), 0)
        lax.fori_loop(0, nchunks // 2, make_f(True), 0)
    else:
      flip = flip_vecs[ks]

      def f(t, _):
        c0 = t * fs
        c = x_vmem[pl.ds(c0, fs), :] * flip
        x_vmem[pl.ds(c0, fs), :] = network(c, strides, False) * flip
        return 0

      lax.fori_loop(0, nchunks, f, 0)

  def pair_pass(size, s):
    p = _log2(s // bc)
    ps = _log2(size // bc)
    nqb = _log2(n_rows // bc)

    def make_f(desc):
      def f(t, _):
        q = _insert_bit(t, p, 0)
        if ps < nqb:
          q = _insert_bit(q, ps, 1 if desc else 0)
        b0 = q * bc
        a = x_vmem[pl.ds(b0, bc), :]
        d = x_vmem[pl.ds(b0 + s, bc), :]
        mn = jnp.minimum(a, d)
        mx = jnp.maximum(a, d)
        x_vmem[pl.ds(b0, bc), :] = mx if desc else mn
        x_vmem[pl.ds(b0 + s, bc), :] = mn if desc else mx
        return 0

      return f

    if ps >= nqb:
      lax.fori_loop(0, n_rows // (2 * bc), make_f(False), 0)
    else:
      lax.fori_loop(0, n_rows // (4 * bc), make_f(False), 0)
      lax.fori_loop(0, n_rows // (4 * bc), make_f(True), 0)

  def quad_pass(size, s1, s2):
    p2 = _log2(s2 // bc)
    ps = _log2(size // bc)
    nqb = _log2(n_rows // bc)

    def make_f(desc):
      lo = jnp.maximum if desc else jnp.minimum
      hi = jnp.minimum if desc else jnp.maximum

      def f(t, _):
        q = _insert_bit(_insert_bit(t, p2, 0), p2 + 1, 0)
        if ps < nqb:
          q = _insert_bit(q, ps, 1 if desc else 0)
        b0 = q * bc
        a = x_vmem[pl.ds(b0, bc), :]
        b_ = x_vmem[pl.ds(b0 + s2, bc), :]
        c = x_vmem[pl.ds(b0 + s1, bc), :]
        d = x_vmem[pl.ds(b0 + s1 + s2, bc), :]
        a1, c1 = lo(a, c), hi(a, c)
        b1, d1 = lo(b_, d), hi(b_, d)
        x_vmem[pl.ds(b0, bc), :] = lo(a1, b1)
        x_vmem[pl.ds(b0 + s2, bc), :] = hi(a1, b1)
        x_vmem[pl.ds(b0 + s1, bc), :] = lo(c1, d1)
        x_vmem[pl.ds(b0 + s1 + s2, bc), :] = hi(c1, d1)
        return 0

      return f

    if ps >= nqb:
      lax.fori_loop(0, n_rows // (4 * bc), make_f(False), 0)
    else:
      lax.fori_loop(0, n_rows // (8 * bc), make_f(False), 0)
      lax.fori_loop(0, n_rows // (8 * bc), make_f(True), 0)

  for ks in range(1, nbits + 1):
    size = 1 << ks
    bigs = [1 << j for j in range(ks - 1, _log2(fs) - 1, -1)]
    i = 0
    while i < len(bigs):
      if i + 1 < len(bigs):
        quad_pass(size, bigs[i], bigs[i + 1])
        i += 2
      else:
        pair_pass(size, bigs[i])
        i += 1
    fused_small_stage(ks)

  rm = min(2048, n_rows)

  def f_mse(t, s):
    c = x_vmem[pl.ds(t * rm, rm), :]
    d = c[:, :cb] - c[:, cb:]
    return s + jnp.sum(d * d)

  total = lax.fori_loop(0, n_rows // rm, f_mse, jnp.float32(0.0))
  o_ref[...] = jnp.full((1, 8, 128), total, jnp.float32)


@jax.jit
def kernel(y_s, y_t):
  n_rows, c = y_s.shape
  assert n_rows & (n_rows - 1) == 0, "rows must be a power of two"
  cb = 64 if c % 64 == 0 else c
  grid = c // cb
  lanes = 2 * cb
  fs = min(128, n_rows)
  bc = min(64, n_rows)
  x_all = jnp.concatenate(
      [y_s.reshape(n_rows, grid, cb), y_t.reshape(n_rows, grid, cb)], axis=2
  ).reshape(n_rows, 2 * c)

  body = functools.partial(
      _sort_mse_body, n_rows=n_rows, lanes=lanes, cb=cb, fs=fs, bc=bc
  )
  partials = pl.pallas_call(
      body,
      grid=(grid,),
      in_specs=[pl.BlockSpec(memory_space=pl.ANY)],
      out_specs=pl.BlockSpec((1, 8, 128), lambda b: (b, 0, 0)),
      out_shape=jax.ShapeDtypeStruct((grid, 8, 128), jnp.float32),
      scratch_shapes=[
          pltpu.VMEM((n_rows, lanes), jnp.float32),
          pltpu.SemaphoreType.DMA,
      ],
      compiler_params=pltpu.CompilerParams(
          dimension_semantics=("arbitrary",),
      ),
  )(x_all)
  return jnp.sum(partials[:, 0, 0]) / (n_rows * c)
